# baseline (speedup 1.0000x reference)
import functools

import jax
import jax.numpy as jnp
from jax import lax
from jax.experimental import pallas as pl
from jax.experimental.pallas import tpu as pltpu
from jax.experimental.pallas import tpu_sc as plsc

DEL = 32
B = 16384
V = 1000000
H = V // 2
NC, NS, L = 2, 16, 16
NW = NC * NS
BPW = B // NW


def _body(idx_hbm, t0_hbm, t1_hbm, out_hbm, idx_v, idx0_v, idx1_v,
          rows0_v, rows1_v, sel_v, sem):
    wid = lax.axis_index("s") * NC + lax.axis_index("c")
    base = wid * BPW
    pltpu.sync_copy(idx_hbm.at[pl.ds(base, BPW)], idx_v)

    def clamp(g, carry):
        r = idx_v[pl.ds(g * L, L)]
        idx0_v[pl.ds(g * L, L)] = jnp.minimum(r, H - 1)
        idx1_v[pl.ds(g * L, L)] = jnp.minimum(jnp.maximum(r - H, 0), H - 1)
        sel_v[pl.ds(g * L, L)] = jnp.where(r < H, 1.0, 0.0).astype(jnp.float32)
        return carry

    lax.fori_loop(0, BPW // L, clamp, 0)
    cp0 = pltpu.async_copy(t0_hbm.at[idx0_v], rows0_v, sem)
    pltpu.async_copy(t1_hbm.at[idx1_v], rows1_v, sem)
    cp0.wait()
    pltpu.make_async_copy(t1_hbm.at[idx1_v], rows1_v, sem).wait()

    def row(g, carry):
        for j in range(L):
            i = g * L + j
            for c in range(DEL // L):
                xa = rows0_v[i, pl.ds(c * L, L)]
                xb = rows1_v[i, pl.ds(c * L, L)]
                m = plsc.load_gather(sel_v, [jnp.full((L,), i, jnp.int32)])
                x = xa * m + xb * (1.0 - m)
                rows0_v[i, pl.ds(c * L, L)] = 1.0 / (1.0 + jnp.exp(-x))
        return carry

    lax.fori_loop(0, BPW // L, row, 0)
    pltpu.sync_copy(rows0_v, out_hbm.at[pl.ds(base, BPW)])


@jax.jit
def _emb_sigmoid(idx, t0, t1):
    mesh = plsc.VectorSubcoreMesh(core_axis_name="c", subcore_axis_name="s")
    f = functools.partial(
        pl.kernel,
        mesh=mesh,
        out_type=jax.ShapeDtypeStruct((B, DEL), jnp.float32),
        scratch_types=[
            pltpu.VMEM((BPW,), jnp.int32),
            pltpu.VMEM((BPW,), jnp.int32),
            pltpu.VMEM((BPW,), jnp.int32),
            pltpu.VMEM((BPW, DEL), jnp.float32),
            pltpu.VMEM((BPW, DEL), jnp.float32),
            pltpu.VMEM((BPW,), jnp.float32),
            pltpu.SemaphoreType.DMA,
        ],
        compiler_params=pltpu.CompilerParams(
            use_tc_tiling_on_sc=False, needs_layout_passes=False
        ),
    )(_body)
    return f(idx, t0, t1)


def kernel(idx, table):
    idx = idx.astype(jnp.int32)
    out = _emb_sigmoid(idx, table[:H], table[H:])
    return out[:, :, None]


# speedup vs baseline: 1.5611x; 1.5611x over previous
import functools

import jax
import jax.numpy as jnp
from jax import lax
from jax.experimental import pallas as pl
from jax.experimental.pallas import tpu as pltpu
from jax.experimental.pallas import tpu_sc as plsc

DEL = 32
B = 16384
NC, NS, L = 2, 16, 16
NW = NC * NS
BPW = B // NW


def _body(idx_hbm, table_hbm, out_hbm, idx_v, rows_v, sem):
    wid = lax.axis_index("s") * NC + lax.axis_index("c")
    base = wid * BPW
    pltpu.sync_copy(idx_hbm.at[pl.ds(base, BPW)], idx_v)
    pltpu.async_copy(table_hbm.at[idx_v], rows_v, sem).wait()

    def row(i, carry):
        for c in range(DEL // L):
            x = rows_v[i, pl.ds(c * L, L)]
            rows_v[i, pl.ds(c * L, L)] = 1.0 / (1.0 + jnp.exp(-x))
        return carry

    lax.fori_loop(0, BPW, row, 0)
    pltpu.sync_copy(rows_v, out_hbm.at[pl.ds(base, BPW)])


@jax.jit
def _emb_sigmoid(idx, table):
    mesh = plsc.VectorSubcoreMesh(core_axis_name="c", subcore_axis_name="s")
    f = functools.partial(
        pl.kernel,
        mesh=mesh,
        out_type=jax.ShapeDtypeStruct((B, DEL), jnp.float32),
        scratch_types=[
            pltpu.VMEM((BPW,), jnp.int32),
            pltpu.VMEM((BPW, DEL), jnp.float32),
            pltpu.SemaphoreType.DMA,
        ],
        compiler_params=pltpu.CompilerParams(use_tc_tiling_on_sc=False),
    )(_body)
    return f(idx, table)


def kernel(idx, table):
    out = _emb_sigmoid(idx.astype(jnp.int32), table)
    return out[:, :, None]
